# baseline (speedup 1.0000x reference)
import jax
import jax.numpy as jnp
from jax.experimental import pallas as pl
from jax.experimental.pallas import tpu as pltpu

_W = 4
_K = 32


def _body(x_hbm, o_hbm, sems):
    n = o_hbm.shape[0]
    rows = n // _K
    cps = []
    for k in range(_K):
        cp = pltpu.make_async_copy(
            x_hbm.at[pl.ds(k * rows, rows), pl.ds(0, 1)],
            o_hbm.at[pl.ds(k * rows, rows)],
            sems.at[k])
        cp.start()
        cps.append(cp)
    for cp in cps:
        cp.wait()


def kernel(x):
    b, s, d = x.shape
    h = s // _W
    n = b * h
    xv = x.reshape(n, _W, d)
    out = pl.pallas_call(
        _body,
        in_specs=[pl.BlockSpec(memory_space=pl.ANY)],
        out_specs=pl.BlockSpec(memory_space=pl.ANY),
        out_shape=jax.ShapeDtypeStruct((n, 1, d), x.dtype),
        scratch_shapes=[pltpu.SemaphoreType.DMA((_K,))],
    )(xv)
    return out.reshape(b, h, d)


# speedup vs baseline: 6.3189x; 6.3189x over previous
import jax
import jax.numpy as jnp
from jax.experimental import pallas as pl
from jax.experimental.pallas import tpu as pltpu

_W = 4
_K = 16


def _body(x_hbm, o_hbm, buf, isems, osems):
    n = o_hbm.shape[0]
    rows = n // _K
    ins = []
    for k in range(_K):
        cp = pltpu.make_async_copy(
            x_hbm.at[pl.ds(k * rows, rows), pl.ds(0, 1)],
            buf.at[pl.ds(k * rows, rows)],
            isems.at[k])
        cp.start()
        ins.append(cp)
    outs = []
    for k in range(_K):
        ins[k].wait()
        cp = pltpu.make_async_copy(
            buf.at[pl.ds(k * rows, rows)],
            o_hbm.at[pl.ds(k * rows, rows)],
            osems.at[k])
        cp.start()
        outs.append(cp)
    for cp in outs:
        cp.wait()


def kernel(x):
    b, s, d = x.shape
    h = s // _W
    n = b * h
    xv = x.reshape(n, _W, d)
    out = pl.pallas_call(
        _body,
        in_specs=[pl.BlockSpec(memory_space=pl.ANY)],
        out_specs=pl.BlockSpec(memory_space=pl.ANY),
        out_shape=jax.ShapeDtypeStruct((n, 1, d), x.dtype),
        scratch_shapes=[
            pltpu.VMEM((n, 1, d), jnp.float32),
            pltpu.SemaphoreType.DMA((_K,)),
            pltpu.SemaphoreType.DMA((_K,)),
        ],
    )(xv)
    return out.reshape(b, h, d)
